# baseline (speedup 1.0000x reference)
import functools

import jax
import jax.numpy as jnp
from jax import lax
from jax.experimental import pallas as pl
from jax.experimental.pallas import tpu as pltpu
from jax.experimental.pallas import tpu_sc as plsc

VOCAB = 1_000_000
EMB = 64
BATCH = 16384

NC = 2
NS = 16
L = 16
NW = NC * NS
BPW = BATCH // NW
NWAVE = BPW // L
IDX_CHUNK = 128
N_CHUNKS = BPW // IDX_CHUNK


def _glove_body(ii_hbm, jj_hbm, wi_hbm, wj_hbm, bi_hbm, bj_hbm, out_hbm,
                idx_i, idx_j, rbi, rbj, bias_i, bias_j, out_v,
                sem0, sem1, semb):
    wid = lax.axis_index("s") * NC + lax.axis_index("c")
    base = wid * BPW

    pltpu.sync_copy(ii_hbm.at[pl.ds(wid * N_CHUNKS, N_CHUNKS)], idx_i)
    pltpu.sync_copy(jj_hbm.at[pl.ds(wid * N_CHUNKS, N_CHUNKS)], idx_j)

    for c in range(N_CHUNKS):
        sl = pl.ds(c * IDX_CHUNK, IDX_CHUNK)
        pltpu.async_copy(bi_hbm.at[idx_i.at[c]], bias_i.at[sl], semb)
        pltpu.async_copy(bj_hbm.at[idx_j.at[c]], bias_j.at[sl], semb)

    lane_id = lax.iota(jnp.int32, L)
    sems = (sem0, sem1)

    def fire(g, p):
        sem = sems[p]
        v_i = jax.lax.shift_right_logical(
            idx_i[g // 8, pl.ds((g % 8) * L, L)], 1)
        v_j = jax.lax.shift_right_logical(
            idx_j[g // 8, pl.ds((g % 8) * L, L)], 1)
        for t in range(L):
            slot = p * L + t
            pltpu.async_copy(wi_hbm.at[pl.ds(v_i[t], 1)],
                             rbi.at[pl.ds(slot, 1)], sem)
            pltpu.async_copy(wj_hbm.at[pl.ds(v_j[t], 1)],
                             rbj.at[pl.ds(slot, 1)], sem)

    def drain(p):
        sem = sems[p]
        pltpu.make_async_copy(wi_hbm.at[pl.ds(0, L)],
                              rbi.at[pl.ds(p * L, L)], sem).wait()
        pltpu.make_async_copy(wj_hbm.at[pl.ds(0, L)],
                              rbj.at[pl.ds(p * L, L)], sem).wait()

    def compute(g, p):
        dots_e = jnp.zeros((L,), jnp.float32)
        dots_o = jnp.zeros((L,), jnp.float32)
        par_i = idx_i[g // 8, pl.ds((g % 8) * L, L)] & 1
        par_j = idx_j[g // 8, pl.ds((g % 8) * L, L)] & 1
        for t in range(L):
            slot = p * L + t
            s_ee = rbi[slot, pl.ds(0, L)] * rbj[slot, pl.ds(0, L)]
            s_eo = rbi[slot, pl.ds(0, L)] * rbj[slot, pl.ds(64, L)]
            s_oe = rbi[slot, pl.ds(64, L)] * rbj[slot, pl.ds(0, L)]
            s_oo = rbi[slot, pl.ds(64, L)] * rbj[slot, pl.ds(64, L)]
            for k in range(1, EMB // L):
                a_e = rbi[slot, pl.ds(k * L, L)]
                a_o = rbi[slot, pl.ds(64 + k * L, L)]
                b_e = rbj[slot, pl.ds(k * L, L)]
                b_o = rbj[slot, pl.ds(64 + k * L, L)]
                s_ee = s_ee + a_e * b_e
                s_eo = s_eo + a_e * b_o
                s_oe = s_oe + a_o * b_e
                s_oo = s_oo + a_o * b_o
            d_ee = jnp.full((L,), jnp.sum(s_ee), jnp.float32)
            d_eo = jnp.full((L,), jnp.sum(s_eo), jnp.float32)
            d_oe = jnp.full((L,), jnp.sum(s_oe), jnp.float32)
            d_oo = jnp.full((L,), jnp.sum(s_oo), jnp.float32)
            here = lane_id == t
            dots_e = jnp.where(here, jnp.where(par_j == 0, d_ee, d_eo),
                               dots_e)
            dots_o = jnp.where(here, jnp.where(par_j == 0, d_oe, d_oo),
                               dots_o)
        dots = jnp.where(par_i == 0, dots_e, dots_o)
        sl = pl.ds(g * L, L)
        out_v[sl] = dots + bias_i[sl] + bias_j[sl]

    fire(0, 0)
    fire(1, 1)

    for c in range(N_CHUNKS):
        sl = pl.ds(c * IDX_CHUNK, IDX_CHUNK)
        pltpu.make_async_copy(bi_hbm.at[pl.ds(0, IDX_CHUNK)],
                              bias_i.at[sl], semb).wait()
        pltpu.make_async_copy(bj_hbm.at[pl.ds(0, IDX_CHUNK)],
                              bias_j.at[sl], semb).wait()

    def step(g2, carry):
        g = g2 * 2
        drain(0)
        compute(g, 0)

        @pl.when(g2 < NWAVE // 2 - 1)
        def _():
            fire(g + 2, 0)

        drain(1)
        compute(g + 1, 1)

        @pl.when(g2 < NWAVE // 2 - 1)
        def _():
            fire(g + 3, 1)

        return carry

    lax.fori_loop(0, NWAVE // 2, step, 0)

    pltpu.sync_copy(out_v, out_hbm.at[pl.ds(base, BPW)])


@jax.jit
def _glove_sc(ii, jj, wi, wj, bi_flat, bj_flat):
    mesh = plsc.VectorSubcoreMesh(core_axis_name="c", subcore_axis_name="s",
                                  num_cores=NC, num_subcores=NS)
    f = functools.partial(
        pl.kernel,
        out_type=jax.ShapeDtypeStruct((BATCH,), jnp.float32),
        mesh=mesh,
        compiler_params=pltpu.CompilerParams(needs_layout_passes=False,
                                             use_tc_tiling_on_sc=True),
        scratch_types=[
            pltpu.VMEM((N_CHUNKS, IDX_CHUNK), jnp.int32),
            pltpu.VMEM((N_CHUNKS, IDX_CHUNK), jnp.int32),
            pltpu.VMEM((2 * L, 2 * EMB), jnp.float32),
            pltpu.VMEM((2 * L, 2 * EMB), jnp.float32),
            pltpu.VMEM((BPW,), jnp.float32),
            pltpu.VMEM((BPW,), jnp.float32),
            pltpu.VMEM((BPW,), jnp.float32),
            pltpu.SemaphoreType.DMA,
            pltpu.SemaphoreType.DMA,
            pltpu.SemaphoreType.DMA,
        ],
    )(_glove_body)
    return f(ii.reshape(NW * N_CHUNKS, IDX_CHUNK),
             jj.reshape(NW * N_CHUNKS, IDX_CHUNK),
             wi, wj, bi_flat, bj_flat)


def kernel(i_indices, j_indices, wi, wj, bi, bj):
    ii = i_indices.astype(jnp.int32)
    jj = j_indices.astype(jnp.int32)
    bif = pltpu.einshape("ab->(ba)", bi, assert_is_tile_preserving=True)
    bjf = pltpu.einshape("ab->(ba)", bj, assert_is_tile_preserving=True)
    wi2 = jnp.reshape(wi, (VOCAB // 2, 2 * EMB))
    wj2 = jnp.reshape(wj, (VOCAB // 2, 2 * EMB))
    return _glove_sc(ii, jj, wi2, wj2, bif, bjf)


# speedup vs baseline: 1.4115x; 1.4115x over previous
import functools

import jax
import jax.numpy as jnp
from jax import lax
from jax.experimental import pallas as pl
from jax.experimental.pallas import tpu as pltpu
from jax.experimental.pallas import tpu_sc as plsc

VOCAB = 1_000_000
EMB = 64
BATCH = 16384

NC = 2
NS = 16
L = 16
NW = NC * NS
BPW = BATCH // NW
NWAVE = BPW // L
IDX_CHUNK = 128
N_CHUNKS = BPW // IDX_CHUNK


def _glove_body(ii_hbm, jj_hbm, wi_hbm, wj_hbm, bi_hbm, bj_hbm, out_hbm,
                idx_i, idx_j, rbi, rbj, bias_i, bias_j, out_v,
                sem0, sem1, semb):
    wid = lax.axis_index("s") * NC + lax.axis_index("c")
    base = wid * BPW

    pltpu.sync_copy(ii_hbm.at[pl.ds(wid * N_CHUNKS, N_CHUNKS)], idx_i)
    pltpu.sync_copy(jj_hbm.at[pl.ds(wid * N_CHUNKS, N_CHUNKS)], idx_j)

    for c in range(N_CHUNKS):
        sl = pl.ds(c * IDX_CHUNK, IDX_CHUNK)
        pltpu.async_copy(bi_hbm.at[idx_i.at[c]], bias_i.at[sl], semb)
        pltpu.async_copy(bj_hbm.at[idx_j.at[c]], bias_j.at[sl], semb)

    lane_id = lax.iota(jnp.int32, L)
    sems = (sem0, sem1)

    def fire(g, p):
        sem = sems[p]
        v_i = idx_i[g // 8, pl.ds((g % 8) * L, L)]
        v_j = idx_j[g // 8, pl.ds((g % 8) * L, L)]
        for t in range(L):
            slot = p * L + t
            pltpu.async_copy(wi_hbm.at[pl.ds(v_i[t], 1)],
                             rbi.at[pl.ds(slot, 1)], sem)
            pltpu.async_copy(wj_hbm.at[pl.ds(v_j[t], 1)],
                             rbj.at[pl.ds(slot, 1)], sem)

    def drain(p):
        sem = sems[p]
        pltpu.make_async_copy(wi_hbm.at[pl.ds(0, L)],
                              rbi.at[pl.ds(p * L, L)], sem).wait()
        pltpu.make_async_copy(wj_hbm.at[pl.ds(0, L)],
                              rbj.at[pl.ds(p * L, L)], sem).wait()

    def compute(g, p):
        dots = jnp.zeros((L,), jnp.float32)
        for t in range(L):
            slot = p * L + t
            s = rbi[slot, pl.ds(0, L)] * rbj[slot, pl.ds(0, L)]
            for k in range(1, EMB // L):
                s = s + rbi[slot, pl.ds(k * L, L)] * rbj[slot, pl.ds(k * L, L)]
            dot = jnp.full((L,), jnp.sum(s), jnp.float32)
            dots = jnp.where(lane_id == t, dot, dots)
        sl = pl.ds(g * L, L)
        out_v[sl] = dots + bias_i[sl] + bias_j[sl]

    fire(0, 0)
    fire(1, 1)

    for c in range(N_CHUNKS):
        sl = pl.ds(c * IDX_CHUNK, IDX_CHUNK)
        pltpu.make_async_copy(bi_hbm.at[pl.ds(0, IDX_CHUNK)],
                              bias_i.at[sl], semb).wait()
        pltpu.make_async_copy(bj_hbm.at[pl.ds(0, IDX_CHUNK)],
                              bias_j.at[sl], semb).wait()

    def step(g2, carry):
        g = g2 * 2
        drain(0)
        compute(g, 0)

        @pl.when(g2 < NWAVE // 2 - 1)
        def _():
            fire(g + 2, 0)

        drain(1)
        compute(g + 1, 1)

        @pl.when(g2 < NWAVE // 2 - 1)
        def _():
            fire(g + 3, 1)

        return carry

    lax.fori_loop(0, NWAVE // 2, step, 0)

    pltpu.sync_copy(out_v, out_hbm.at[pl.ds(base, BPW)])


@jax.jit
def _glove_sc(ii, jj, wi, wj, bi_flat, bj_flat):
    mesh = plsc.VectorSubcoreMesh(core_axis_name="c", subcore_axis_name="s",
                                  num_cores=NC, num_subcores=NS)
    f = functools.partial(
        pl.kernel,
        out_type=jax.ShapeDtypeStruct((BATCH,), jnp.float32),
        mesh=mesh,
        compiler_params=pltpu.CompilerParams(needs_layout_passes=False,
                                             use_tc_tiling_on_sc=True),
        scratch_types=[
            pltpu.VMEM((N_CHUNKS, IDX_CHUNK), jnp.int32),
            pltpu.VMEM((N_CHUNKS, IDX_CHUNK), jnp.int32),
            pltpu.VMEM((2 * L, EMB), jnp.float32),
            pltpu.VMEM((2 * L, EMB), jnp.float32),
            pltpu.VMEM((BPW,), jnp.float32),
            pltpu.VMEM((BPW,), jnp.float32),
            pltpu.VMEM((BPW,), jnp.float32),
            pltpu.SemaphoreType.DMA,
            pltpu.SemaphoreType.DMA,
            pltpu.SemaphoreType.DMA,
        ],
    )(_glove_body)
    return f(ii.reshape(NW * N_CHUNKS, IDX_CHUNK),
             jj.reshape(NW * N_CHUNKS, IDX_CHUNK),
             wi, wj, bi_flat, bj_flat)


def kernel(i_indices, j_indices, wi, wj, bi, bj):
    ii = i_indices.astype(jnp.int32)
    jj = j_indices.astype(jnp.int32)
    bif = pltpu.einshape("ab->(ba)", bi, assert_is_tile_preserving=True)
    bjf = pltpu.einshape("ab->(ba)", bj, assert_is_tile_preserving=True)
    return _glove_sc(ii, jj, wi, wj, bif, bjf)
